# baseline (speedup 1.0000x reference)
import functools

import jax
import jax.numpy as jnp
from jax import lax
from jax.experimental import pallas as pl
from jax.experimental.pallas import tpu as pltpu
from jax.experimental.pallas import tpu_sc as plsc

_B = 1024
_H = 20
_D = 1000
_DP = 1024
_N = _B * _H
_NW = 32
_LPW = _N // _NW
_G = 32
_STEPS = _LPW // _G
_R = 512


def _make_sc_gather():
    mesh = plsc.VectorSubcoreMesh(core_axis_name="c", subcore_axis_name="s")

    @functools.partial(
        pl.kernel,
        mesh=mesh,
        out_type=jax.ShapeDtypeStruct((_N, _DP), jnp.float32),
        scratch_types=[
            pltpu.VMEM((_LPW,), jnp.int32),
            pltpu.VMEM((2, _G, _DP), jnp.float32),
            pltpu.SemaphoreType.DMA,
            pltpu.SemaphoreType.DMA,
        ],
    )
    def gather(table_hbm, idx_hbm, out_hbm, idx_v, rows_v, gsem, ssem):
        wid = lax.axis_index("s") * 2 + lax.axis_index("c")
        base = wid * _LPW
        pltpu.sync_copy(idx_hbm.at[pl.ds(base, _LPW)], idx_v)

        def gstart(i, b):
            pltpu.async_copy(
                table_hbm.at[idx_v.at[pl.ds(i * _G, _G)]], rows_v.at[b], gsem
            )

        def gwait():
            pltpu.make_async_copy(
                table_hbm.at[idx_v.at[pl.ds(0, _G)]], rows_v.at[0], gsem
            ).wait()

        def sstart(i, b):
            pltpu.async_copy(
                rows_v.at[b], out_hbm.at[pl.ds(base + i * _G, _G)], ssem
            )

        def swait():
            pltpu.make_async_copy(
                rows_v.at[0], out_hbm.at[pl.ds(base, _G)], ssem
            ).wait()

        gstart(0, 0)

        def body(i, carry):
            b = lax.rem(i, 2)

            @pl.when(i + 1 < _STEPS)
            def _():
                @pl.when(i >= 1)
                def _():
                    swait()

                gstart(i + 1, 1 - b)

            gwait()
            sstart(i, b)
            return carry

        lax.fori_loop(0, _STEPS, body, 0)
        swait()
        swait()

    return gather


_sc_gather = _make_sc_gather()


def _pad_body(x_ref, o_ref):
    o_ref[...] = jnp.pad(x_ref[...], ((0, 0), (0, _DP - _D)))


_pad_table = pl.pallas_call(
    _pad_body,
    grid=(1,),
    in_specs=[pl.BlockSpec((1000, _D), lambda i: (0, 0))],
    out_specs=pl.BlockSpec((1000, _DP), lambda i: (0, 0)),
    out_shape=jax.ShapeDtypeStruct((1000, _DP), jnp.float32),
)


def _repack_body(x_ref, o_ref):
    o_ref[...] = x_ref[:, :, : _D]


_BR = 32


_repack = pl.pallas_call(
    _repack_body,
    grid=(_B // _BR,),
    in_specs=[pl.BlockSpec((_BR, _H, _DP), lambda i: (i, 0, 0))],
    out_specs=pl.BlockSpec((_BR, _H, _D), lambda i: (i, 0, 0)),
    out_shape=jax.ShapeDtypeStruct((_B, _H, _D), jnp.float32),
)


def kernel(knowledge, table):
    padded = _sc_gather(_pad_table(table), knowledge.reshape(_N))
    return _repack(padded.reshape(_B, _H, _DP))


# speedup vs baseline: 1.1019x; 1.1019x over previous
import functools

import jax
import jax.numpy as jnp
from jax import lax
from jax.experimental import pallas as pl
from jax.experimental.pallas import tpu as pltpu
from jax.experimental.pallas import tpu_sc as plsc

_B = 1024
_H = 20
_D = 1000
_DP = 1024
_N = _B * _H
_NW = 32
_LPW = _N // _NW
_G = 32
_STEPS = _LPW // _G
_R = 512


def _make_sc_gather():
    mesh = plsc.VectorSubcoreMesh(core_axis_name="c", subcore_axis_name="s")

    @functools.partial(
        pl.kernel,
        mesh=mesh,
        out_type=jax.ShapeDtypeStruct((_N, _DP), jnp.float32),
        scratch_types=[
            pltpu.VMEM((_LPW,), jnp.int32),
            pltpu.VMEM((2, _G, _DP), jnp.float32),
            pltpu.SemaphoreType.DMA,
            pltpu.SemaphoreType.DMA,
        ],
    )
    def gather(table_hbm, idx_hbm, out_hbm, idx_v, rows_v, gsem, ssem):
        wid = lax.axis_index("s") * 2 + lax.axis_index("c")
        base = wid * _LPW
        pltpu.sync_copy(idx_hbm.at[pl.ds(base, _LPW)], idx_v)

        def gstart(i, b):
            pltpu.async_copy(
                table_hbm.at[idx_v.at[pl.ds(i * _G, _G)]], rows_v.at[b], gsem
            )

        def gwait():
            pltpu.make_async_copy(
                table_hbm.at[idx_v.at[pl.ds(0, _G)]], rows_v.at[0], gsem
            ).wait()

        def sstart(i, b):
            pltpu.async_copy(
                rows_v.at[b], out_hbm.at[pl.ds(base + i * _G, _G)], ssem
            )

        def swait():
            pltpu.make_async_copy(
                rows_v.at[0], out_hbm.at[pl.ds(base, _G)], ssem
            ).wait()

        gstart(0, 0)

        def body(i, carry):
            b = lax.rem(i, 2)

            @pl.when(i + 1 < _STEPS)
            def _():
                @pl.when(i >= 1)
                def _():
                    swait()

                gstart(i + 1, 1 - b)

            gwait()
            sstart(i, b)
            return carry

        lax.fori_loop(0, _STEPS, body, 0)
        swait()
        swait()

    return gather


_sc_gather = _make_sc_gather()


def _pad_body(x_ref, o_ref):
    o_ref[...] = jnp.pad(x_ref[...], ((0, 0), (0, _DP - _D)))


_pad_table = pl.pallas_call(
    _pad_body,
    grid=(1,),
    in_specs=[pl.BlockSpec((1000, _D), lambda i: (0, 0))],
    out_specs=pl.BlockSpec((1000, _DP), lambda i: (0, 0)),
    out_shape=jax.ShapeDtypeStruct((1000, _DP), jnp.float32),
)


def _repack_body(x_ref, o_ref):
    o_ref[...] = x_ref[:, : _D]


_repack = pl.pallas_call(
    _repack_body,
    grid=(_N // _R,),
    in_specs=[pl.BlockSpec((_R, _DP), lambda i: (i, 0))],
    out_specs=pl.BlockSpec((_R, _D), lambda i: (i, 0)),
    out_shape=jax.ShapeDtypeStruct((_N, _D), jnp.float32),
)


def kernel(knowledge, table):
    padded = _sc_gather(_pad_table(table), knowledge.reshape(_N))
    return _repack(padded).reshape(_B, _H, _D)
